# baseline (speedup 1.0000x reference)
import functools

import jax
import jax.numpy as jnp
from jax import lax
from jax.experimental import pallas as pl
from jax.experimental.pallas import tpu as pltpu
from jax.experimental.pallas import tpu_sc as plsc

N = 10000
D = 128
E = 320000
NC = 2
NS = 16
CH = 128
ROWS_PER_TILE = 160
ROWS = ROWS_PER_TILE * NS
E_PAD = ROWS * CH
NODES_PER_SC = 5120
ACC_ROWS = 5136
DUMMY = NODES_PER_SC
OUT_ROWS = 2 * NODES_PER_SC
ZERO_PER_TILE = ACC_ROWS // NS
OUT_PER_TILE = NODES_PER_SC // NS
NBUF = 3

_BM = 1000


def _pre_body(x1_ref, x2_ref, w_ref, b_ref, o1_ref, o2_ref):
    w = w_ref[...]
    b = b_ref[...]
    o1_ref[...] = jnp.dot(x1_ref[...], w, preferred_element_type=jnp.float32) + b
    o2_ref[...] = jnp.dot(x2_ref[...], w, preferred_element_type=jnp.float32) + b


_pre_call = pl.pallas_call(
    _pre_body,
    grid=(N // _BM,),
    in_specs=[
        pl.BlockSpec((_BM, D), lambda i: (i, 0)),
        pl.BlockSpec((_BM, D), lambda i: (i, 0)),
        pl.BlockSpec((D, D), lambda i: (0, 0)),
        pl.BlockSpec((1, D), lambda i: (0, 0)),
    ],
    out_specs=[pl.BlockSpec((_BM, D), lambda i: (i, 0))] * 2,
    out_shape=[jax.ShapeDtypeStruct((N, D), jnp.float32)] * 2,
)


def _sc_body(pre1_h, pre2_h, src_h, dst_h, out1_h, out2_h,
             src_v, dst_v, sbuf, gbuf, acc,
             gsem0, gsem1, gsem2, ssem0, ssem1, ssem2):
    cid = lax.axis_index("c")
    sid = lax.axis_index("s")
    lo = cid * NODES_PER_SC
    gsems = (gsem0, gsem1, gsem2)
    ssems = (ssem0, ssem1, ssem2)
    R = ROWS_PER_TILE
    WPT = R * CH // 2

    pltpu.sync_copy(src_h.at[pl.ds(sid * WPT, WPT)], src_v)
    pltpu.sync_copy(dst_h.at[pl.ds(sid * ROWS_PER_TILE, ROWS_PER_TILE)], dst_v)

    def _remap(i, carry):
        r = i // (CH // 16)
        c = i % (CH // 16)
        d = dst_v[r, pl.ds(c * 16, 16)]
        m = (d >= lo) & (d < lo + NODES_PER_SC)
        dst_v[r, pl.ds(c * 16, 16)] = jnp.where(m, d - lo, DUMMY + (d & 15))
        return carry

    lax.fori_loop(0, ROWS_PER_TILE * (CH // 16), _remap, None)

    def _stage(r):
        base = (r & 3) * CH
        for q in range(CH // 32):
            w = src_v[pl.ds(r * (CH // 2) + q * 16, 16)]
            sbuf[pl.ds(base + q * 32, 16)] = w & 0xFFFF
            sbuf[pl.ds(base + q * 32 + 16, 16)] = lax.shift_right_logical(w, 16)

    def _start_g(pre_h, r, b):
        idx = sbuf.at[pl.ds((r & 3) * CH, CH)]
        pltpu.async_copy(pre_h.at[idx], gbuf.at[b], gsems[b])

    def _wait_g(pre_h, r, b):
        idx = sbuf.at[pl.ds((r & 3) * CH, CH)]
        pltpu.make_async_copy(pre_h.at[idx], gbuf.at[b], gsems[b]).wait()

    def _start_s(r, b):
        pltpu.async_copy(gbuf.at[b], acc.at[dst_v.at[r]], ssems[b], add=True)

    def _wait_s(r, b):
        pltpu.make_async_copy(gbuf.at[b], acc.at[dst_v.at[r]],
                              ssems[b]).wait()

    def _process(pre_h):
        for r in range(NBUF):
            _stage(r)
            _start_g(pre_h, r, r)
        _wait_g(pre_h, 0, 0)
        _start_s(0, 0)

        def _loop(j, carry):
            r0 = 1 + j * NBUF
            for u in range(NBUF):
                r = r0 + u
                b = (1 + u) % NBUF
                _wait_g(pre_h, r, b)
                _start_s(r, b)
                _wait_s(r - 1, u)
                _stage(r + 2)
                _start_g(pre_h, r + 2, u)
            return carry

        lax.fori_loop(0, (R - 4) // NBUF, _loop, None)
        r = R - 3
        _wait_g(pre_h, r, r % NBUF)
        _start_s(r, r % NBUF)
        _wait_s(r - 1, (r - 1) % NBUF)
        _stage(R - 1)
        _start_g(pre_h, R - 1, (R - 1) % NBUF)
        for r in (R - 2, R - 1):
            _wait_g(pre_h, r, r % NBUF)
            _start_s(r, r % NBUF)
            _wait_s(r - 1, (r - 1) % NBUF)
        _wait_s(R - 1, (R - 1) % NBUF)

    pres = (pre1_h, pre2_h)
    outs = (out1_h, out2_h)
    zrow = sid * ZERO_PER_TILE
    row0 = sid * OUT_PER_TILE
    g0 = gbuf.at[0]
    for phase in range(2):
        def _zero(i, carry):
            r = i // (D // 16)
            c = i % (D // 16)
            g0[r, pl.ds(c * 16, 16)] = jnp.zeros((16,), jnp.float32)
            return carry

        lax.fori_loop(0, CH * (D // 16), _zero, None)
        pltpu.sync_copy(g0, acc.at[pl.ds(zrow, CH)])
        pltpu.sync_copy(g0, acc.at[pl.ds(zrow + CH, CH)])
        pltpu.sync_copy(g0.at[pl.ds(0, ZERO_PER_TILE - 2 * CH - 1)],
                        acc.at[pl.ds(zrow + 2 * CH, ZERO_PER_TILE - 2 * CH - 1)])
        pltpu.sync_copy(g0.at[pl.ds(0, 1)],
                        acc.at[pl.ds(zrow + ZERO_PER_TILE - 1, 1)])
        plsc.subcore_barrier()

        _process(pres[phase])

        plsc.subcore_barrier()

        pltpu.sync_copy(acc.at[pl.ds(row0, OUT_PER_TILE)],
                        outs[phase].at[pl.ds(lo + row0, OUT_PER_TILE)])

        plsc.subcore_barrier()


@functools.cache
def _sc_call():
    return pl.kernel(
        _sc_body,
        out_type=[jax.ShapeDtypeStruct((OUT_ROWS, D), jnp.float32)] * 2,
        mesh=plsc.VectorSubcoreMesh(core_axis_name="c", subcore_axis_name="s"),
        scratch_types=[
            pltpu.VMEM((ROWS_PER_TILE * CH // 2,), jnp.int32),
            pltpu.VMEM((ROWS_PER_TILE, CH), jnp.int32),
            pltpu.VMEM((4 * CH,), jnp.int32),
            pltpu.VMEM((NBUF, CH, D), jnp.float32),
            pltpu.VMEM_SHARED((ACC_ROWS, D), jnp.float32),
            pltpu.SemaphoreType.DMA,
            pltpu.SemaphoreType.DMA,
            pltpu.SemaphoreType.DMA,
            pltpu.SemaphoreType.DMA,
            pltpu.SemaphoreType.DMA,
            pltpu.SemaphoreType.DMA,
        ],
    )


def _disc_body(a1_ref, a2_ref, wd_ref, bd_ref, o1_ref, o2_ref):
    valid = lax.broadcasted_iota(jnp.int32, (OUT_ROWS, 1), 0) < N
    h1 = a1_ref[...]
    h1 = jnp.where(valid, jnp.where(h1 > 0, h1, 0.25 * h1), 0.0)
    h2 = a2_ref[...]
    h2 = jnp.where(h2 > 0, h2, 0.25 * h2)
    c = jax.nn.sigmoid(jnp.sum(h1, axis=0, keepdims=True) / N)
    wc = lax.dot_general(c, wd_ref[...], (((1,), (1,)), ((), ())))
    b = bd_ref[0, 0]
    o1_ref[...] = jnp.sum(h1 * wc, axis=1, keepdims=True) + b
    o2_ref[...] = jnp.sum(h2 * wc, axis=1, keepdims=True) + b


_disc_call = pl.pallas_call(
    _disc_body,
    out_shape=[jax.ShapeDtypeStruct((OUT_ROWS, 1), jnp.float32)] * 2,
)


def kernel(seq1, seq2, adj, W_gcn, b_gcn, W_disc, b_disc):
    pre1, pre2 = _pre_call(seq1, seq2, W_gcn, b_gcn.reshape(1, D))
    pad = E_PAD - E
    sp = jnp.concatenate([adj[0], jnp.zeros((pad,), jnp.int32)])
    sp = sp.reshape(E_PAD // 32, 2, 16)
    src = (sp[:, 0, :] | (sp[:, 1, :] << 16)).reshape(E_PAD // 2)
    dst = jnp.concatenate([adj[1], jnp.full((pad,), N, jnp.int32)]).reshape(ROWS, CH)
    agg1, agg2 = _sc_call()(pre1, pre2, src, dst)
    o1, o2 = _disc_call(agg1, agg2, W_disc, b_disc.reshape(1, 1))
    return jnp.concatenate([o1[:N, 0], o2[:N, 0]])


# speedup vs baseline: 1.4781x; 1.4781x over previous
import functools

import jax
import jax.numpy as jnp
from jax import lax
from jax.experimental import pallas as pl
from jax.experimental.pallas import tpu as pltpu
from jax.experimental.pallas import tpu_sc as plsc

N = 10000
D = 128
E = 320000
NC = 2
NS = 16
CH = 128
ROWS_PER_TILE = 160
ROWS = ROWS_PER_TILE * NS
E_PAD = ROWS * CH
NODES_PER_SC = 5120
ACC_ROWS = 5120
OUT_ROWS = 2 * NODES_PER_SC
ZERO_PER_TILE = ACC_ROWS // NS
OUT_PER_TILE = NODES_PER_SC // NS
NBUF = 2

_BM = 1000


def _pre_body(x1_ref, x2_ref, w_ref, b_ref, o1_ref, o2_ref):
    w = w_ref[...]
    b = b_ref[...]
    o1_ref[...] = jnp.dot(x1_ref[...], w, preferred_element_type=jnp.float32) + b
    o2_ref[...] = jnp.dot(x2_ref[...], w, preferred_element_type=jnp.float32) + b


_pre_call = pl.pallas_call(
    _pre_body,
    grid=(N // _BM,),
    in_specs=[
        pl.BlockSpec((_BM, D), lambda i: (i, 0)),
        pl.BlockSpec((_BM, D), lambda i: (i, 0)),
        pl.BlockSpec((D, D), lambda i: (0, 0)),
        pl.BlockSpec((1, D), lambda i: (0, 0)),
    ],
    out_specs=[pl.BlockSpec((_BM, D), lambda i: (i, 0))] * 2,
    out_shape=[jax.ShapeDtypeStruct((N, D), jnp.float32)] * 2,
)


def _sc_body(pre1_h, pre2_h, src_h, dst_h, out1_h, out2_h,
             src_v, dst_v, gbuf, acc, gsem0, gsem1):
    cid = lax.axis_index("c")
    sid = lax.axis_index("s")
    lo = cid * NODES_PER_SC
    gsems = (gsem0, gsem1)
    R = ROWS_PER_TILE
    EPT = R * CH

    pltpu.sync_copy(src_h.at[pl.ds(sid * EPT, EPT)], src_v)
    pltpu.sync_copy(dst_h.at[pl.ds(sid * R, R)], dst_v)

    def _remap(i, carry):
        r = i // (CH // 16)
        c = i % (CH // 16)
        d = dst_v[r, pl.ds(c * 16, 16)]
        s = src_v[pl.ds(i * 16, 16)]
        m = (d >= lo) & (d < lo + NODES_PER_SC)
        dst_v[r, pl.ds(c * 16, 16)] = jnp.where(m, d - lo, -1)
        src_v[pl.ds(i * 16, 16)] = jnp.where(m, s, -1)
        return carry

    lax.fori_loop(0, R * (CH // 16), _remap, None)

    def _gidx(r):
        return plsc.Indices(src_v.at[pl.ds(r * CH, CH)], ignored_value=-1)

    def _start_g(pre_h, r, b):
        pltpu.async_copy(pre_h.at[_gidx(r)], gbuf.at[b], gsems[b])

    def _wait_g(pre_h, r, b):
        pltpu.make_async_copy(pre_h.at[_gidx(r)], gbuf.at[b], gsems[b]).wait()

    def _process(pre_h):
        _start_g(pre_h, 0, 0)
        _start_g(pre_h, 1, 1)

        def _loop(j, carry):
            for u in range(2):
                r = j * 2 + u
                _wait_g(pre_h, r, u)
                pltpu.sync_copy(
                    gbuf.at[u],
                    acc.at[plsc.Indices(dst_v.at[r], ignored_value=-1)],
                    add=True)
                _start_g(pre_h, r + 2, u)
            return carry

        lax.fori_loop(0, (R - 2) // 2, _loop, None)
        for u in range(2):
            r = R - 2 + u
            _wait_g(pre_h, r, u)
            pltpu.sync_copy(
                gbuf.at[u],
                acc.at[plsc.Indices(dst_v.at[r], ignored_value=-1)],
                add=True)

    pres = (pre1_h, pre2_h)
    outs = (out1_h, out2_h)
    zrow = sid * ZERO_PER_TILE
    row0 = sid * OUT_PER_TILE
    g0 = gbuf.at[0]
    for phase in range(2):
        def _zero(i, carry):
            r = i // (D // 16)
            c = i % (D // 16)
            g0[r, pl.ds(c * 16, 16)] = jnp.zeros((16,), jnp.float32)
            return carry

        lax.fori_loop(0, CH * (D // 16), _zero, None)
        pltpu.sync_copy(g0, acc.at[pl.ds(zrow, CH)])
        pltpu.sync_copy(g0, acc.at[pl.ds(zrow + CH, CH)])
        pltpu.sync_copy(g0.at[pl.ds(0, ZERO_PER_TILE - 2 * CH)],
                        acc.at[pl.ds(zrow + 2 * CH, ZERO_PER_TILE - 2 * CH)])
        plsc.subcore_barrier()

        _process(pres[phase])

        plsc.subcore_barrier()

        pltpu.sync_copy(acc.at[pl.ds(row0, OUT_PER_TILE)],
                        outs[phase].at[pl.ds(lo + row0, OUT_PER_TILE)])

        plsc.subcore_barrier()

@functools.cache
def _sc_call():
    return pl.kernel(
        _sc_body,
        out_type=[jax.ShapeDtypeStruct((OUT_ROWS, D), jnp.float32)] * 2,
        mesh=plsc.VectorSubcoreMesh(core_axis_name="c", subcore_axis_name="s"),
        scratch_types=[
            pltpu.VMEM((ROWS_PER_TILE * CH,), jnp.int32),
            pltpu.VMEM((ROWS_PER_TILE, CH), jnp.int32),
            pltpu.VMEM((NBUF, CH, D), jnp.float32),
            pltpu.VMEM_SHARED((ACC_ROWS, D), jnp.float32),
            pltpu.SemaphoreType.DMA,
            pltpu.SemaphoreType.DMA,
        ],
    )


def _disc_body(a1_ref, a2_ref, wd_ref, bd_ref, o1_ref, o2_ref):
    valid = lax.broadcasted_iota(jnp.int32, (OUT_ROWS, 1), 0) < N
    h1 = a1_ref[...]
    h1 = jnp.where(valid, jnp.where(h1 > 0, h1, 0.25 * h1), 0.0)
    h2 = a2_ref[...]
    h2 = jnp.where(h2 > 0, h2, 0.25 * h2)
    c = jax.nn.sigmoid(jnp.sum(h1, axis=0, keepdims=True) / N)
    wc = lax.dot_general(c, wd_ref[...], (((1,), (1,)), ((), ())))
    b = bd_ref[0, 0]
    o1_ref[...] = jnp.sum(h1 * wc, axis=1, keepdims=True) + b
    o2_ref[...] = jnp.sum(h2 * wc, axis=1, keepdims=True) + b


_disc_call = pl.pallas_call(
    _disc_body,
    out_shape=[jax.ShapeDtypeStruct((OUT_ROWS, 1), jnp.float32)] * 2,
)


def kernel(seq1, seq2, adj, W_gcn, b_gcn, W_disc, b_disc):
    pre1, pre2 = _pre_call(seq1, seq2, W_gcn, b_gcn.reshape(1, D))
    pad = E_PAD - E
    src = jnp.concatenate([adj[0], jnp.zeros((pad,), jnp.int32)])
    dst = jnp.concatenate([adj[1], jnp.full((pad,), N, jnp.int32)]).reshape(ROWS, CH)
    agg1, agg2 = _sc_call()(pre1, pre2, src, dst)
    o1, o2 = _disc_call(agg1, agg2, W_disc, b_disc.reshape(1, 1))
    return jnp.concatenate([o1[:N, 0], o2[:N, 0]])


# speedup vs baseline: 1.5129x; 1.0236x over previous
import functools

import jax
import jax.numpy as jnp
from jax import lax
from jax.experimental import pallas as pl
from jax.experimental.pallas import tpu as pltpu
from jax.experimental.pallas import tpu_sc as plsc

N = 10000
D = 128
E = 320000
NC = 2
NS = 16
CH = 128
ROWS_PER_TILE = 160
ROWS = ROWS_PER_TILE * NS
E_PAD = ROWS * CH
NODES_PER_SC = 5120
ACC_ROWS = 5120
OUT_ROWS = 2 * NODES_PER_SC
ZERO_PER_TILE = ACC_ROWS // NS
OUT_PER_TILE = NODES_PER_SC // NS
NBUF = 3

_BM = 1000


def _pre_body(x1_ref, x2_ref, w_ref, b_ref, o1_ref, o2_ref):
    w = w_ref[...]
    b = b_ref[...]
    o1_ref[...] = jnp.dot(x1_ref[...], w, preferred_element_type=jnp.float32) + b
    o2_ref[...] = jnp.dot(x2_ref[...], w, preferred_element_type=jnp.float32) + b


_pre_call = pl.pallas_call(
    _pre_body,
    grid=(N // _BM,),
    in_specs=[
        pl.BlockSpec((_BM, D), lambda i: (i, 0)),
        pl.BlockSpec((_BM, D), lambda i: (i, 0)),
        pl.BlockSpec((D, D), lambda i: (0, 0)),
        pl.BlockSpec((1, D), lambda i: (0, 0)),
    ],
    out_specs=[pl.BlockSpec((_BM, D), lambda i: (i, 0))] * 2,
    out_shape=[jax.ShapeDtypeStruct((N, D), jnp.float32)] * 2,
)


def _sc_body(pre1_h, pre2_h, src_h, dst_h, out1_h, out2_h,
             src_v, dst_v, sbuf, gbuf, acc,
             gsem0, gsem1, gsem2, ssem0, ssem1, ssem2):
    cid = lax.axis_index("c")
    sid = lax.axis_index("s")
    lo = cid * NODES_PER_SC
    gsems = (gsem0, gsem1, gsem2)
    ssems = (ssem0, ssem1, ssem2)
    R = ROWS_PER_TILE
    WPT = R * CH // 2

    pltpu.sync_copy(src_h.at[pl.ds(sid * WPT, WPT)], src_v)
    pltpu.sync_copy(dst_h.at[pl.ds(sid * R, R)], dst_v)

    def _remap(i, carry):
        r = i // 4
        c = (i % 4) * 32
        d0 = dst_v[r, pl.ds(c, 16)]
        d1 = dst_v[r, pl.ds(c + 16, 16)]
        m0 = (d0 >= lo) & (d0 < lo + NODES_PER_SC)
        m1 = (d1 >= lo) & (d1 < lo + NODES_PER_SC)
        dst_v[r, pl.ds(c, 16)] = jnp.where(m0, d0 - lo, -1)
        dst_v[r, pl.ds(c + 16, 16)] = jnp.where(m1, d1 - lo, -1)
        w = src_v[pl.ds(i * 16, 16)]
        s0 = jnp.where(m0, w & 0xFFFF, 0xFFFF)
        s1 = jnp.where(m1, lax.shift_right_logical(w, 16), 0xFFFF)
        src_v[pl.ds(i * 16, 16)] = s0 | lax.shift_left(s1, 16)
        return carry

    lax.fori_loop(0, WPT // 16, _remap, None)

    def _stage(r):
        base = (r & 3) * CH
        for q in range(CH // 32):
            w = src_v[pl.ds(r * (CH // 2) + q * 16, 16)]
            sbuf[pl.ds(base + q * 32, 16)] = w & 0xFFFF
            sbuf[pl.ds(base + q * 32 + 16, 16)] = lax.shift_right_logical(w, 16)

    def _gidx(r):
        return plsc.Indices(sbuf.at[pl.ds((r & 3) * CH, CH)],
                            ignored_value=0xFFFF)

    def _sidx(r):
        return plsc.Indices(dst_v.at[r], ignored_value=-1)

    def _start_g(pre_h, r, b):
        pltpu.async_copy(pre_h.at[_gidx(r)], gbuf.at[b], gsems[b])

    def _wait_g(pre_h, r, b):
        pltpu.make_async_copy(pre_h.at[_gidx(r)], gbuf.at[b], gsems[b]).wait()

    def _start_s(r, b):
        pltpu.async_copy(gbuf.at[b], acc.at[_sidx(r)], ssems[b], add=True)

    def _wait_s(r, b):
        pltpu.make_async_copy(gbuf.at[b], acc.at[_sidx(r)], ssems[b]).wait()

    def _process(pre_h):
        for r in range(NBUF):
            _stage(r)
            _start_g(pre_h, r, r)

        def _loop(j, carry):
            r0 = j * NBUF
            for u in range(NBUF):
                r = r0 + u
                _wait_g(pre_h, r, u)
                pltpu.sync_copy(gbuf.at[u], acc.at[_sidx(r)], add=True)
                _stage(r + NBUF)
                _start_g(pre_h, r + NBUF, u)
            return carry

        lax.fori_loop(0, (R - 4) // NBUF, _loop, None)
        for k, r in enumerate(range(R - 4, R)):
            b = r % NBUF
            _wait_g(pre_h, r, b)
            pltpu.sync_copy(gbuf.at[b], acc.at[_sidx(r)], add=True)
            if k == 0:
                _stage(R - 1)
                _start_g(pre_h, R - 1, b)

    pres = (pre1_h, pre2_h)
    outs = (out1_h, out2_h)
    zrow = sid * ZERO_PER_TILE
    row0 = sid * OUT_PER_TILE
    g0 = gbuf.at[0]
    for phase in range(2):
        def _zero(i, carry):
            r = i // (D // 16)
            c = i % (D // 16)
            g0[r, pl.ds(c * 16, 16)] = jnp.zeros((16,), jnp.float32)
            return carry

        lax.fori_loop(0, CH * (D // 16), _zero, None)
        pltpu.sync_copy(g0, acc.at[pl.ds(zrow, CH)])
        pltpu.sync_copy(g0, acc.at[pl.ds(zrow + CH, CH)])
        pltpu.sync_copy(g0.at[pl.ds(0, ZERO_PER_TILE - 2 * CH)],
                        acc.at[pl.ds(zrow + 2 * CH, ZERO_PER_TILE - 2 * CH)])
        plsc.subcore_barrier()

        _process(pres[phase])

        plsc.subcore_barrier()

        pltpu.sync_copy(acc.at[pl.ds(row0, OUT_PER_TILE)],
                        outs[phase].at[pl.ds(lo + row0, OUT_PER_TILE)])

        plsc.subcore_barrier()

@functools.cache
def _sc_call():
    return pl.kernel(
        _sc_body,
        out_type=[jax.ShapeDtypeStruct((OUT_ROWS, D), jnp.float32)] * 2,
        mesh=plsc.VectorSubcoreMesh(core_axis_name="c", subcore_axis_name="s"),
        scratch_types=[
            pltpu.VMEM((ROWS_PER_TILE * CH // 2,), jnp.int32),
            pltpu.VMEM((ROWS_PER_TILE, CH), jnp.int32),
            pltpu.VMEM((4 * CH,), jnp.int32),
            pltpu.VMEM((NBUF, CH, D), jnp.float32),
            pltpu.VMEM_SHARED((ACC_ROWS, D), jnp.float32),
            pltpu.SemaphoreType.DMA,
            pltpu.SemaphoreType.DMA,
            pltpu.SemaphoreType.DMA,
            pltpu.SemaphoreType.DMA,
            pltpu.SemaphoreType.DMA,
            pltpu.SemaphoreType.DMA,
        ],
    )


def _disc_body(a1_ref, a2_ref, wd_ref, bd_ref, o1_ref, o2_ref):
    valid = lax.broadcasted_iota(jnp.int32, (OUT_ROWS, 1), 0) < N
    h1 = a1_ref[...]
    h1 = jnp.where(valid, jnp.where(h1 > 0, h1, 0.25 * h1), 0.0)
    h2 = a2_ref[...]
    h2 = jnp.where(h2 > 0, h2, 0.25 * h2)
    c = jax.nn.sigmoid(jnp.sum(h1, axis=0, keepdims=True) / N)
    wc = lax.dot_general(c, wd_ref[...], (((1,), (1,)), ((), ())))
    b = bd_ref[0, 0]
    o1_ref[...] = jnp.sum(h1 * wc, axis=1, keepdims=True) + b
    o2_ref[...] = jnp.sum(h2 * wc, axis=1, keepdims=True) + b


_disc_call = pl.pallas_call(
    _disc_body,
    out_shape=[jax.ShapeDtypeStruct((OUT_ROWS, 1), jnp.float32)] * 2,
)


def kernel(seq1, seq2, adj, W_gcn, b_gcn, W_disc, b_disc):
    pre1, pre2 = _pre_call(seq1, seq2, W_gcn, b_gcn.reshape(1, D))
    pad = E_PAD - E
    sp = jnp.concatenate([adj[0], jnp.zeros((pad,), jnp.int32)])
    sp = sp.reshape(E_PAD // 32, 2, 16)
    src = (sp[:, 0, :] | (sp[:, 1, :] << 16)).reshape(E_PAD // 2)
    dst = jnp.concatenate([adj[1], jnp.full((pad,), N, jnp.int32)]).reshape(ROWS, CH)
    agg1, agg2 = _sc_call()(pre1, pre2, src, dst)
    o1, o2 = _disc_call(agg1, agg2, W_disc, b_disc.reshape(1, 1))
    return jnp.concatenate([o1[:N, 0], o2[:N, 0]])


# speedup vs baseline: 4.1730x; 2.7583x over previous
import functools

import jax
import jax.numpy as jnp
from jax import lax
from jax.experimental import pallas as pl
from jax.experimental.pallas import tpu as pltpu
from jax.experimental.pallas import tpu_sc as plsc

N = 10000
D = 128
E = 320000
NC = 2
NS = 16
CH = 128
ROWS_PER_TILE = 160
ROWS = ROWS_PER_TILE * NS
E_PAD = ROWS * CH
NODES_PER_SC = 5120
ACC_ROWS = 5120
OUT_ROWS = 2 * NODES_PER_SC
ZERO_PER_TILE = ACC_ROWS // NS
OUT_PER_TILE = NODES_PER_SC // NS
NBUF = 3

_BM = 1000


def _pre_body(x1_ref, x2_ref, w_ref, b_ref, o1_ref, o2_ref):
    w = w_ref[...]
    b = b_ref[...]
    o1_ref[...] = jnp.dot(x1_ref[...], w, preferred_element_type=jnp.float32) + b
    o2_ref[...] = jnp.dot(x2_ref[...], w, preferred_element_type=jnp.float32) + b


_pre_call = pl.pallas_call(
    _pre_body,
    grid=(N // _BM,),
    in_specs=[
        pl.BlockSpec((_BM, D), lambda i: (i, 0)),
        pl.BlockSpec((_BM, D), lambda i: (i, 0)),
        pl.BlockSpec((D, D), lambda i: (0, 0)),
        pl.BlockSpec((1, D), lambda i: (0, 0)),
    ],
    out_specs=[pl.BlockSpec((_BM, D), lambda i: (i, 0))] * 2,
    out_shape=[jax.ShapeDtypeStruct((N, D), jnp.float32)] * 2,
)


def _sc_body(pre1_h, pre2_h, src_h, dst_h, out1_h, out2_h,
             src_v, dst_v, sbuf, gbuf, acc,
             gsem0, gsem1, gsem2, ssem0, ssem1, ssem2):
    cid = lax.axis_index("c")
    sid = lax.axis_index("s")
    lo = cid * NODES_PER_SC
    gsems = (gsem0, gsem1, gsem2)
    ssems = (ssem0, ssem1, ssem2)
    R = ROWS_PER_TILE
    WPT = R * CH // 2

    pltpu.sync_copy(src_h.at[pl.ds(sid * WPT, WPT)], src_v)
    pltpu.sync_copy(dst_h.at[pl.ds(sid * R, R)], dst_v)

    def _remap(i, carry):
        r = i // 4
        c = (i % 4) * 32
        d0 = dst_v[r, pl.ds(c, 16)]
        d1 = dst_v[r, pl.ds(c + 16, 16)]
        m0 = (d0 >= lo) & (d0 < lo + NODES_PER_SC)
        m1 = (d1 >= lo) & (d1 < lo + NODES_PER_SC)
        dst_v[r, pl.ds(c, 16)] = jnp.where(m0, d0 - lo, -1)
        dst_v[r, pl.ds(c + 16, 16)] = jnp.where(m1, d1 - lo, -1)
        w = src_v[pl.ds(i * 16, 16)]
        s0 = jnp.where(m0, w & 0xFFFF, 0xFFFF)
        s1 = jnp.where(m1, lax.shift_right_logical(w, 16), 0xFFFF)
        src_v[pl.ds(i * 16, 16)] = s0 | lax.shift_left(s1, 16)
        return carry

    lax.fori_loop(0, WPT // 16, _remap, None)

    def _stage(r):
        base = (r & 3) * CH
        for q in range(CH // 32):
            w = src_v[pl.ds(r * (CH // 2) + q * 16, 16)]
            sbuf[pl.ds(base + q * 32, 16)] = w & 0xFFFF
            sbuf[pl.ds(base + q * 32 + 16, 16)] = lax.shift_right_logical(w, 16)

    def _gidx(r):
        return plsc.Indices(sbuf.at[pl.ds((r & 3) * CH, CH)],
                            ignored_value=0xFFFF)

    def _sidx(r):
        return plsc.Indices(dst_v.at[r], ignored_value=-1)

    def _start_g(pre_h, r, b):
        pltpu.async_copy(pre_h.at[_gidx(r)], gbuf.at[b], gsems[b])

    def _wait_g(pre_h, r, b):
        pltpu.make_async_copy(pre_h.at[_gidx(r)], gbuf.at[b], gsems[b]).wait()

    def _start_s(r, b):
        pltpu.async_copy(gbuf.at[b], acc.at[_sidx(r)], ssems[b], add=True)

    def _wait_s(r, b):
        pltpu.make_async_copy(gbuf.at[b], acc.at[_sidx(r)], ssems[b]).wait()

    def _process(pre_h):
        for r in range(NBUF):
            _stage(r)
            _start_g(pre_h, r, r)

        def _loop(j, carry):
            r0 = j * NBUF
            for u in range(NBUF):
                r = r0 + u
                _wait_g(pre_h, r, u)
                pltpu.sync_copy(gbuf.at[u], acc.at[_sidx(r)], add=True)
                _stage(r + NBUF)
                _start_g(pre_h, r + NBUF, u)
            return carry

        lax.fori_loop(0, (R - 4) // NBUF, _loop, None)
        for k, r in enumerate(range(R - 4, R)):
            b = r % NBUF
            _wait_g(pre_h, r, b)
            pltpu.sync_copy(gbuf.at[b], acc.at[_sidx(r)], add=True)
            if k == 0:
                _stage(R - 1)
                _start_g(pre_h, R - 1, b)

    pres = (pre1_h, pre2_h)
    outs = (out1_h, out2_h)
    zrow = sid * ZERO_PER_TILE
    row0 = sid * OUT_PER_TILE
    g0 = gbuf.at[0]
    for phase in range(2):
        def _zero(i, carry):
            r = i // (D // 16)
            c = i % (D // 16)
            g0[r, pl.ds(c * 16, 16)] = jnp.zeros((16,), jnp.float32)
            return carry

        lax.fori_loop(0, CH * (D // 16), _zero, None)
        pltpu.sync_copy(g0, acc.at[pl.ds(zrow, CH)])
        pltpu.sync_copy(g0, acc.at[pl.ds(zrow + CH, CH)])
        pltpu.sync_copy(g0.at[pl.ds(0, ZERO_PER_TILE - 2 * CH)],
                        acc.at[pl.ds(zrow + 2 * CH, ZERO_PER_TILE - 2 * CH)])
        plsc.subcore_barrier()

        _process(pres[phase])

        plsc.subcore_barrier()

        pltpu.sync_copy(acc.at[pl.ds(row0, OUT_PER_TILE)],
                        outs[phase].at[pl.ds(lo + row0, OUT_PER_TILE)])

        plsc.subcore_barrier()

@functools.cache
def _sc_call():
    return pl.kernel(
        _sc_body,
        out_type=[jax.ShapeDtypeStruct((OUT_ROWS, D), jnp.float32)] * 2,
        mesh=plsc.VectorSubcoreMesh(core_axis_name="c", subcore_axis_name="s"),
        scratch_types=[
            pltpu.VMEM((ROWS_PER_TILE * CH // 2,), jnp.int32),
            pltpu.VMEM((ROWS_PER_TILE, CH), jnp.int32),
            pltpu.VMEM((4 * CH,), jnp.int32),
            pltpu.VMEM((NBUF, CH, D), jnp.float32),
            pltpu.VMEM_SHARED((ACC_ROWS, D), jnp.float32),
            pltpu.SemaphoreType.DMA,
            pltpu.SemaphoreType.DMA,
            pltpu.SemaphoreType.DMA,
            pltpu.SemaphoreType.DMA,
            pltpu.SemaphoreType.DMA,
            pltpu.SemaphoreType.DMA,
        ],
    )


def _disc_body(a1_ref, a2_ref, wd_ref, bd_ref, o1_ref, o2_ref):
    valid = lax.broadcasted_iota(jnp.int32, (OUT_ROWS, 1), 0) < N
    h1 = a1_ref[...]
    h1 = jnp.where(valid, jnp.where(h1 > 0, h1, 0.25 * h1), 0.0)
    h2 = a2_ref[...]
    h2 = jnp.where(h2 > 0, h2, 0.25 * h2)
    c = jax.nn.sigmoid(jnp.sum(h1, axis=0, keepdims=True) / N)
    wc = lax.dot_general(c, wd_ref[...], (((1,), (1,)), ((), ())))
    b = bd_ref[0, 0]
    o1_ref[...] = jnp.sum(h1 * wc, axis=1, keepdims=True) + b
    o2_ref[...] = jnp.sum(h2 * wc, axis=1, keepdims=True) + b


_disc_call = pl.pallas_call(
    _disc_body,
    out_shape=[jax.ShapeDtypeStruct((OUT_ROWS, 1), jnp.float32)] * 2,
)


def kernel(seq1, seq2, adj, W_gcn, b_gcn, W_disc, b_disc):
    pre1, pre2 = _pre_call(seq1, seq2, W_gcn, b_gcn.reshape(1, D))
    pad = E_PAD - E
    sp = jnp.concatenate([adj[0], jnp.zeros((pad,), jnp.int32)])
    sp = sp.reshape(E_PAD // 32, 2, 16)
    src = (sp[:, 0, :] | (sp[:, 1, :] << 16)).reshape(E_PAD // 2)
    dst = jnp.concatenate([adj[1], jnp.full((pad,), 2 * N, jnp.int32)]).reshape(ROWS, CH)
    agg1, agg2 = _sc_call()(pre1, pre2, src, dst)
    o1, o2 = _disc_call(agg1, agg2, W_disc, b_disc.reshape(1, 1))
    return jnp.concatenate([o1[:N, 0], o2[:N, 0]])
